# baseline (speedup 1.0000x reference)
import jax
import jax.numpy as jnp
from jax import lax
from jax.experimental import pallas as pl
from jax.experimental.pallas import tpu as pltpu
from jax.experimental.pallas import tpu_sc as plsc

P = 1024
D = 512
M = 65536
TOP_K = 50
N = P * TOP_K
MB = 2048

NC = 2
NS = 16
NW = NC * NS

M_PER_TILE = M // NS
N_PER_W = N // NW
FLAT_CHUNK = N // NS
STAGE_PAD = 256
STAGE = N + STAGE_PAD
ZCHUNK = STAGE // NS
GROWS = 64
NGC = N_PER_W // GROWS


def _sim_block(lhs_ref, rhs_ref, out_ref):
    out_ref[...] = jax.lax.dot_general(
        lhs_ref[...], rhs_ref[...],
        dimension_numbers=(((1,), (1,)), ((), ())),
        preferred_element_type=jnp.float32,
    )


def _similarity(test_patches, memory_nodes):
    return pl.pallas_call(
        _sim_block,
        grid=(M // MB,),
        in_specs=[
            pl.BlockSpec((P, D), lambda j: (0, 0)),
            pl.BlockSpec((MB, D), lambda j: (j, 0)),
        ],
        out_specs=pl.BlockSpec((P, MB), lambda j: (0, j)),
        out_shape=jax.ShapeDtypeStruct((P, M), jnp.float32),
    )(test_patches, memory_nodes)


def _unique_gather_kernel(flat_hbm, mem_hbm, inv_hbm, active_hbm,
                          ranks_sh, stage_sh, totals_sh,
                          flat_v, present_v, ranks_v, scat_idx, scat_val,
                          zeros_v, tot_v, inv_src, inv_out, gidx, gbuf, sem):
    c = lax.axis_index("c")
    s = lax.axis_index("s")
    w = c * NS + s
    lanes = lax.iota(jnp.int32, 16)
    ones16 = jnp.full((16,), 1, jnp.int32)
    zeros16 = jnp.zeros((16,), jnp.int32)
    m_base = s * M_PER_TILE

    def _zero(i, _):
        present_v[pl.ds(i * 16, 16)] = zeros16
        return 0
    lax.fori_loop(0, M_PER_TILE // 16, _zero, 0)

    def _zero2(i, _):
        zeros_v[pl.ds(i * 16, 16)] = zeros16
        return 0
    lax.fori_loop(0, ZCHUNK // 16, _zero2, 0)

    def _scan_chunk(ch, _):
        pltpu.sync_copy(flat_hbm.at[pl.ds(ch * FLAT_CHUNK, FLAT_CHUNK)], flat_v)
        def _mark(i, _):
            v = flat_v[pl.ds(i * 16, 16)]
            rel = v - m_base
            msk = (rel >= 0) & (rel < M_PER_TILE)
            plsc.store_scatter(present_v, [rel], ones16, mask=msk)
            return 0
        lax.fori_loop(0, FLAT_CHUNK // 16, _mark, 0)
        return 0
    lax.fori_loop(0, NS, _scan_chunk, 0)

    def _csum(i, carry):
        x = present_v[pl.ds(i * 16, 16)]
        inc = plsc.cumsum(x)
        ranks_v[pl.ds(i * 16, 16)] = inc - x + carry
        return carry + jnp.max(inc)
    total = lax.fori_loop(0, M_PER_TILE // 16, _csum, jnp.int32(0))
    flat_v[pl.ds(0, 16)] = jnp.full((16,), 1, jnp.int32) * total
    pltpu.sync_copy(flat_v.at[pl.ds(0, 16)], totals_sh.at[pl.ds(s * 16, 16)])
    plsc.subcore_barrier()

    pltpu.sync_copy(totals_sh, tot_v)
    off = jnp.int32(0)
    for r in range(NS):
        row = tot_v[pl.ds(r * 16, 16)]
        off = off + jnp.where(jnp.int32(r) < s, row[0], 0)

    def _shift(i, _):
        ranks_v[pl.ds(i * 16, 16)] = ranks_v[pl.ds(i * 16, 16)] + off
        return 0
    lax.fori_loop(0, M_PER_TILE // 16, _shift, 0)
    pltpu.sync_copy(ranks_v, ranks_sh.at[pl.ds(m_base, M_PER_TILE)])
    pltpu.sync_copy(zeros_v, stage_sh.at[pl.ds(s * ZCHUNK, ZCHUNK)])
    plsc.subcore_barrier()

    def _build(i, _):
        pr = present_v[pl.ds(i * 16, 16)]
        rk = ranks_v[pl.ds(i * 16, 16)]
        mv = m_base + i * 16 + lanes
        dump = N + lax.rem(i, jnp.int32(13)) * 16 + lanes
        idx = jnp.where(pr > 0, rk, dump)
        row = i // 8
        col = (i % 8) * 16
        scat_idx[row, pl.ds(col, 16)] = idx
        scat_val[row, pl.ds(col, 16)] = mv
        return 0
    lax.fori_loop(0, M_PER_TILE // 16, _build, 0)
    for j in range(M_PER_TILE // 128):
        pltpu.sync_copy(scat_val.at[j], stage_sh.at[scat_idx.at[j]])
    plsc.subcore_barrier()

    pltpu.sync_copy(flat_hbm.at[pl.ds(w * N_PER_W, N_PER_W)], inv_src)
    for j in range(N_PER_W // 80):
        pltpu.sync_copy(ranks_sh.at[inv_src.at[pl.ds(j * 80, 80)]],
                        inv_out.at[pl.ds(j * 80, 80)])
    pltpu.sync_copy(inv_out, inv_hbm.at[pl.ds(w * N_PER_W, N_PER_W)])

    pltpu.sync_copy(stage_sh.at[pl.ds(w * N_PER_W, N_PER_W)], gidx)
    def _gchunk(j, _):
        pltpu.async_copy(mem_hbm.at[gidx.at[pl.ds(j * GROWS, GROWS)]], gbuf, sem).wait()
        pltpu.sync_copy(gbuf, active_hbm.at[pl.ds(w * N_PER_W + j * GROWS, GROWS), :])
        return 0
    lax.fori_loop(0, NGC, _gchunk, 0)


def _unique_gather(flat, memory_nodes):
    mesh = plsc.VectorSubcoreMesh(core_axis_name="c", subcore_axis_name="s",
                                  num_cores=NC, num_subcores=NS)
    return pl.kernel(
        _unique_gather_kernel,
        out_type=(
            jax.ShapeDtypeStruct((N,), jnp.int32),
            jax.ShapeDtypeStruct((N, D), jnp.float32),
        ),
        mesh=mesh,
        compiler_params=pltpu.CompilerParams(needs_layout_passes=False),
        scratch_types=[
            pltpu.VMEM_SHARED((M,), jnp.int32),
            pltpu.VMEM_SHARED((STAGE,), jnp.int32),
            pltpu.VMEM_SHARED((NS * 16,), jnp.int32),
            pltpu.VMEM((FLAT_CHUNK,), jnp.int32),
            pltpu.VMEM((M_PER_TILE,), jnp.int32),
            pltpu.VMEM((M_PER_TILE,), jnp.int32),
            pltpu.VMEM((M_PER_TILE // 128, 128), jnp.int32),
            pltpu.VMEM((M_PER_TILE // 128, 128), jnp.int32),
            pltpu.VMEM((ZCHUNK,), jnp.int32),
            pltpu.VMEM((NS * 16,), jnp.int32),
            pltpu.VMEM((N_PER_W,), jnp.int32),
            pltpu.VMEM((N_PER_W,), jnp.int32),
            pltpu.VMEM((N_PER_W,), jnp.int32),
            pltpu.VMEM((GROWS, D), jnp.float32),
            pltpu.SemaphoreType.DMA,
        ],
    )(flat, memory_nodes)


def kernel(test_patches, memory_nodes_gpu):
    sim = _similarity(test_patches, memory_nodes_gpu)
    _, topk_idx = jax.lax.top_k(sim, TOP_K)
    flat = topk_idx.reshape(-1).astype(jnp.int32)
    inverse, active = _unique_gather(flat, memory_nodes_gpu)
    test_node_idx = jnp.repeat(jnp.arange(P, dtype=jnp.int32), TOP_K)
    edge_index = jnp.stack([inverse, test_node_idx], axis=0)
    return edge_index, active


# speedup vs baseline: 4.6760x; 4.6760x over previous
import jax
import jax.numpy as jnp
from jax import lax
from jax.experimental import pallas as pl
from jax.experimental.pallas import tpu as pltpu
from jax.experimental.pallas import tpu_sc as plsc

P = 1024
D = 512
M = 65536
TOP_K = 50
N = P * TOP_K
MB = 2048

NC = 2
NS = 16
NW = NC * NS

M_PER_TILE = M // NS
N_PER_W = N // NW
FLAT_CHUNK = N // NS
STAGE_PAD = 256
STAGE = N + STAGE_PAD
ZCHUNK = STAGE // NS
GROWS = 64
NGC = N_PER_W // GROWS


GS = 128
NG = M // GS
GPB = MB // GS
NBLK = M // MB
NEG = -3.4e38
CAP = 128
RPW = P // NW


def _sim_block(lhs_ref, rhs_ref, out_ref, gmax_ref):
    s = jax.lax.dot_general(
        lhs_ref[...], rhs_ref[...],
        dimension_numbers=(((1,), (1,)), ((), ())),
        preferred_element_type=jnp.float32,
    )
    out_ref[...] = s
    gm = jnp.max(s.reshape(P, GPB, GS), axis=2)
    pad = jnp.full((P, 128 - GPB), NEG, jnp.float32)
    gmax_ref[...] = jnp.concatenate([gm, pad], axis=1)


def _similarity(test_patches, memory_nodes):
    return pl.pallas_call(
        _sim_block,
        grid=(NBLK,),
        in_specs=[
            pl.BlockSpec((P, D), lambda j: (0, 0)),
            pl.BlockSpec((MB, D), lambda j: (j, 0)),
        ],
        out_specs=[
            pl.BlockSpec((P, MB), lambda j: (0, j)),
            pl.BlockSpec((P, 128), lambda j: (0, j)),
        ],
        out_shape=[
            jax.ShapeDtypeStruct((P, M), jnp.float32),
            jax.ShapeDtypeStruct((P, NBLK * 128), jnp.float32),
        ],
    )(test_patches, memory_nodes)


def _thresh_block(gmax_ref, out_ref):
    w = gmax_ref[...]

    def _iter(i, w):
        m = jnp.max(w, axis=1, keepdims=True)
        return jnp.where(w == m, NEG, w)
    w = lax.fori_loop(0, TOP_K - 1, _iter, w)
    t = jnp.max(w, axis=1, keepdims=True)
    out_ref[...] = jnp.broadcast_to(t, (P, 128))


def _threshold(gmax_pad):
    return pl.pallas_call(
        _thresh_block,
        grid=(1,),
        in_specs=[pl.BlockSpec((P, NBLK * 128), lambda j: (0, 0))],
        out_specs=pl.BlockSpec((P, 128), lambda j: (0, 0)),
        out_shape=jax.ShapeDtypeStruct((P, 128), jnp.float32),
    )(gmax_pad)


def _topk_kernel(simv_hbm, gmax_hbm, th_hbm, topk_hbm,
                 grow_v, tv, gids_v, gdma_v, sbuf, cand_v, cand_i, orow_v, sem):
    c = lax.axis_index("c")
    s = lax.axis_index("s")
    w = c * NS + s
    lanes = lax.iota(jnp.int32, 16)
    r0 = w * RPW

    pltpu.sync_copy(th_hbm.at[pl.ds(r0, RPW), :], tv)

    def _row(rl, _):
        r = r0 + rl
        pltpu.sync_copy(gmax_hbm.at[r], grow_v)
        trow = tv[rl, pl.ds(0, 16)]
        t = trow[0]

        for k in range(8):
            gids_v[pl.ds(k * 16, 16)] = k * 16 + lanes
        qcnt = jnp.int32(0)
        for k in range(NG // 16):
            g = grow_v[pl.ds(k * 128, 16)]
            q = g >= t
            qi = jnp.where(q, 1, 0).astype(jnp.int32)
            inc = plsc.cumsum(qi)
            pos = qcnt + inc - qi
            msk = q & (pos < CAP)
            plsc.store_scatter(gids_v, [pos], k * 16 + lanes, mask=msk)
            qcnt = qcnt + jnp.max(inc)
        qcnt = jnp.minimum(qcnt, jnp.int32(CAP))

        def _gd(k, _):
            gv = gids_v[pl.ds(k * 16, 16)]
            gdma_v[pl.ds(k * 16, 16)] = r * NG + gv
            return 0
        lax.fori_loop(0, 8, _gd, 0)
        pltpu.async_copy(simv_hbm.at[gdma_v], sbuf, sem).wait()

        def _zc(k, _):
            cand_v[pl.ds(k * 16, 16)] = jnp.full((16,), NEG, jnp.float32)
            cand_i[pl.ds(k * 16, 16)] = jnp.full((16,), 0x7FFFFFFF, jnp.int32)
            return 0
        lax.fori_loop(0, CAP // 16, _zc, 0)

        nslot8 = (qcnt + 7) // 8

        def _ext(j8, ccnt):
            gwin = gids_v[pl.ds(j8 * 8, 16)]
            for jr in range(8):
                slot = j8 * 8 + jr
                valid = slot < qcnt
                gsc = gwin[jr]
                for u in range(8):
                    v = sbuf[slot, pl.ds(u * 16, 16)]
                    mk = (v >= t) & valid
                    mi = jnp.where(mk, 1, 0).astype(jnp.int32)
                    inc = plsc.cumsum(mi)
                    pos = ccnt + inc - mi
                    mk2 = mk & (pos < CAP)
                    gl = gsc * GS + u * 16 + lanes
                    plsc.store_scatter(cand_v, [pos], v, mask=mk2)
                    plsc.store_scatter(cand_i, [pos], gl, mask=mk2)
                    ccnt = ccnt + jnp.max(inc)
            return ccnt
        ccnt = lax.fori_loop(0, nslot8, _ext, jnp.int32(0))
        ccnt = jnp.minimum(ccnt, jnp.int32(CAP))

        nj8 = (ccnt + 7) // 8

        def _rank(j8, rks):
            vwin = cand_v[pl.ds(j8 * 8, 16)]
            iwin = cand_i[pl.ds(j8 * 8, 16)]
            for jr in range(8):
                vj = vwin[jr]
                ij = iwin[jr]
                nr = []
                for u in range(8):
                    vi = cand_v[pl.ds(u * 16, 16)]
                    ii = cand_i[pl.ds(u * 16, 16)]
                    beat = (vj > vi) | ((vj == vi) & (ij < ii))
                    nr.append(rks[u] + jnp.where(beat, 1, 0).astype(jnp.int32))
                rks = tuple(nr)
            return rks
        rks = lax.fori_loop(0, nj8, _rank,
                            tuple(jnp.zeros((16,), jnp.int32) for _ in range(8)))

        def _zo(k, _):
            orow_v[pl.ds(k * 16, 16)] = jnp.zeros((16,), jnp.int32)
            return 0
        lax.fori_loop(0, 4, _zo, 0)
        for u in range(8):
            ii = cand_i[pl.ds(u * 16, 16)]
            rk = rks[u]
            plsc.store_scatter(orow_v, [rk], ii, mask=rk < TOP_K)
        pltpu.sync_copy(orow_v, topk_hbm.at[r])
        return 0
    lax.fori_loop(0, RPW, _row, 0)


def _topk_sc(simv, gmax_pad, thresh_b):
    mesh = plsc.VectorSubcoreMesh(core_axis_name="c", subcore_axis_name="s",
                                  num_cores=NC, num_subcores=NS)
    return pl.kernel(
        _topk_kernel,
        out_type=jax.ShapeDtypeStruct((P, 64), jnp.int32),
        mesh=mesh,
        compiler_params=pltpu.CompilerParams(needs_layout_passes=False),
        scratch_types=[
            pltpu.VMEM((NBLK * 128,), jnp.float32),
            pltpu.VMEM((RPW, 128), jnp.float32),
            pltpu.VMEM((CAP,), jnp.int32),
            pltpu.VMEM((CAP,), jnp.int32),
            pltpu.VMEM((CAP, GS), jnp.float32),
            pltpu.VMEM((CAP,), jnp.float32),
            pltpu.VMEM((CAP,), jnp.int32),
            pltpu.VMEM((64,), jnp.int32),
            pltpu.SemaphoreType.DMA,
        ],
    )(simv, gmax_pad, thresh_b)


def _unique_gather_kernel(flat_hbm, mem_hbm, inv_hbm, active_hbm,
                          ranks_sh, stage_sh, totals_sh,
                          flat_v, present_v, ranks_v, scat_idx, scat_val,
                          zeros_v, tot_v, inv_src, inv_out, gidx, gbuf, sem):
    c = lax.axis_index("c")
    s = lax.axis_index("s")
    w = c * NS + s
    lanes = lax.iota(jnp.int32, 16)
    ones16 = jnp.full((16,), 1, jnp.int32)
    zeros16 = jnp.zeros((16,), jnp.int32)
    m_base = s * M_PER_TILE

    def _zero(i, _):
        present_v[pl.ds(i * 16, 16)] = zeros16
        return 0
    lax.fori_loop(0, M_PER_TILE // 16, _zero, 0)

    def _zero2(i, _):
        zeros_v[pl.ds(i * 16, 16)] = zeros16
        return 0
    lax.fori_loop(0, ZCHUNK // 16, _zero2, 0)

    def _scan_chunk(ch, _):
        pltpu.sync_copy(flat_hbm.at[pl.ds(ch * FLAT_CHUNK, FLAT_CHUNK)], flat_v)
        def _mark(i, _):
            v = flat_v[pl.ds(i * 16, 16)]
            rel = v - m_base
            msk = (rel >= 0) & (rel < M_PER_TILE)
            plsc.store_scatter(present_v, [rel], ones16, mask=msk)
            return 0
        lax.fori_loop(0, FLAT_CHUNK // 16, _mark, 0)
        return 0
    lax.fori_loop(0, NS, _scan_chunk, 0)

    def _csum(i, carry):
        x = present_v[pl.ds(i * 16, 16)]
        inc = plsc.cumsum(x)
        ranks_v[pl.ds(i * 16, 16)] = inc - x + carry
        return carry + jnp.max(inc)
    total = lax.fori_loop(0, M_PER_TILE // 16, _csum, jnp.int32(0))
    flat_v[pl.ds(0, 16)] = jnp.full((16,), 1, jnp.int32) * total
    pltpu.sync_copy(flat_v.at[pl.ds(0, 16)], totals_sh.at[pl.ds(s * 16, 16)])
    plsc.subcore_barrier()

    pltpu.sync_copy(totals_sh, tot_v)
    off = jnp.int32(0)
    for r in range(NS):
        row = tot_v[pl.ds(r * 16, 16)]
        off = off + jnp.where(jnp.int32(r) < s, row[0], 0)

    def _shift(i, _):
        ranks_v[pl.ds(i * 16, 16)] = ranks_v[pl.ds(i * 16, 16)] + off
        return 0
    lax.fori_loop(0, M_PER_TILE // 16, _shift, 0)
    pltpu.sync_copy(ranks_v, ranks_sh.at[pl.ds(m_base, M_PER_TILE)])
    pltpu.sync_copy(zeros_v, stage_sh.at[pl.ds(s * ZCHUNK, ZCHUNK)])
    plsc.subcore_barrier()

    def _build(i, _):
        pr = present_v[pl.ds(i * 16, 16)]
        rk = ranks_v[pl.ds(i * 16, 16)]
        mv = m_base + i * 16 + lanes
        dump = N + lax.rem(i, jnp.int32(13)) * 16 + lanes
        idx = jnp.where(pr > 0, rk, dump)
        row = i // 8
        col = (i % 8) * 16
        scat_idx[row, pl.ds(col, 16)] = idx
        scat_val[row, pl.ds(col, 16)] = mv
        return 0
    lax.fori_loop(0, M_PER_TILE // 16, _build, 0)
    for j in range(M_PER_TILE // 128):
        pltpu.sync_copy(scat_val.at[j], stage_sh.at[scat_idx.at[j]])
    plsc.subcore_barrier()

    pltpu.sync_copy(flat_hbm.at[pl.ds(w * N_PER_W, N_PER_W)], inv_src)
    for j in range(N_PER_W // 80):
        pltpu.sync_copy(ranks_sh.at[inv_src.at[pl.ds(j * 80, 80)]],
                        inv_out.at[pl.ds(j * 80, 80)])
    pltpu.sync_copy(inv_out, inv_hbm.at[pl.ds(w * N_PER_W, N_PER_W)])

    pltpu.sync_copy(stage_sh.at[pl.ds(w * N_PER_W, N_PER_W)], gidx)
    def _gchunk(j, _):
        pltpu.async_copy(mem_hbm.at[gidx.at[pl.ds(j * GROWS, GROWS)]], gbuf, sem).wait()
        pltpu.sync_copy(gbuf, active_hbm.at[pl.ds(w * N_PER_W + j * GROWS, GROWS), :])
        return 0
    lax.fori_loop(0, NGC, _gchunk, 0)


def _unique_gather(flat, memory_nodes):
    mesh = plsc.VectorSubcoreMesh(core_axis_name="c", subcore_axis_name="s",
                                  num_cores=NC, num_subcores=NS)
    return pl.kernel(
        _unique_gather_kernel,
        out_type=(
            jax.ShapeDtypeStruct((N,), jnp.int32),
            jax.ShapeDtypeStruct((N, D), jnp.float32),
        ),
        mesh=mesh,
        compiler_params=pltpu.CompilerParams(needs_layout_passes=False),
        scratch_types=[
            pltpu.VMEM_SHARED((M,), jnp.int32),
            pltpu.VMEM_SHARED((STAGE,), jnp.int32),
            pltpu.VMEM_SHARED((NS * 16,), jnp.int32),
            pltpu.VMEM((FLAT_CHUNK,), jnp.int32),
            pltpu.VMEM((M_PER_TILE,), jnp.int32),
            pltpu.VMEM((M_PER_TILE,), jnp.int32),
            pltpu.VMEM((M_PER_TILE // 128, 128), jnp.int32),
            pltpu.VMEM((M_PER_TILE // 128, 128), jnp.int32),
            pltpu.VMEM((ZCHUNK,), jnp.int32),
            pltpu.VMEM((NS * 16,), jnp.int32),
            pltpu.VMEM((N_PER_W,), jnp.int32),
            pltpu.VMEM((N_PER_W,), jnp.int32),
            pltpu.VMEM((N_PER_W,), jnp.int32),
            pltpu.VMEM((GROWS, D), jnp.float32),
            pltpu.SemaphoreType.DMA,
        ],
    )(flat, memory_nodes)


def kernel(test_patches, memory_nodes_gpu):
    sim, gmax_pad = _similarity(test_patches, memory_nodes_gpu)
    thresh_b = _threshold(gmax_pad)
    simv = sim.reshape(P * NG, GS)
    topk_pad = _topk_sc(simv, gmax_pad, thresh_b)
    flat = topk_pad[:, :TOP_K].reshape(-1)
    inverse, active = _unique_gather(flat, memory_nodes_gpu)
    test_node_idx = jnp.repeat(jnp.arange(P, dtype=jnp.int32), TOP_K)
    edge_index = jnp.stack([inverse, test_node_idx], axis=0)
    return edge_index, active


# speedup vs baseline: 4.8070x; 1.0280x over previous
import jax
import jax.numpy as jnp
from jax import lax
from jax.experimental import pallas as pl
from jax.experimental.pallas import tpu as pltpu
from jax.experimental.pallas import tpu_sc as plsc

P = 1024
D = 512
M = 65536
TOP_K = 50
N = P * TOP_K
MB = 2048

NC = 2
NS = 16
NW = NC * NS

M_PER_TILE = M // NS
N_PER_W = N // NW
FLAT_CHUNK = N // NS
STAGE_PAD = 256
STAGE = N + STAGE_PAD
ZCHUNK = STAGE // NS
GROWS = 64
NGC = N_PER_W // GROWS


GS = 128
NG = M // GS
GPB = MB // GS
NBLK = M // MB
NEG = -3.4e38
CAP = 128
RPW = P // NW


def _sim_block(lhs_ref, rhs_ref, out_ref, gmax_ref):
    s = jax.lax.dot_general(
        lhs_ref[...], rhs_ref[...],
        dimension_numbers=(((1,), (1,)), ((), ())),
        preferred_element_type=jnp.float32,
    )
    out_ref[...] = s
    gm = jnp.max(s.reshape(P, GPB, GS), axis=2)
    pad = jnp.full((P, 128 - GPB), NEG, jnp.float32)
    gmax_ref[...] = jnp.concatenate([gm, pad], axis=1)


def _similarity(test_patches, memory_nodes):
    return pl.pallas_call(
        _sim_block,
        grid=(NBLK,),
        in_specs=[
            pl.BlockSpec((P, D), lambda j: (0, 0)),
            pl.BlockSpec((MB, D), lambda j: (j, 0)),
        ],
        out_specs=[
            pl.BlockSpec((P, MB), lambda j: (0, j)),
            pl.BlockSpec((P, 128), lambda j: (0, j)),
        ],
        out_shape=[
            jax.ShapeDtypeStruct((P, M), jnp.float32),
            jax.ShapeDtypeStruct((P, NBLK * 128), jnp.float32),
        ],
    )(test_patches, memory_nodes)


def _thresh_block(gmax_ref, out_ref):
    w = gmax_ref[...]

    def _iter(i, w):
        m = jnp.max(w, axis=1, keepdims=True)
        return jnp.where(w == m, NEG, w)
    w = lax.fori_loop(0, TOP_K - 1, _iter, w)
    t = jnp.max(w, axis=1, keepdims=True)
    out_ref[...] = jnp.broadcast_to(t, (P, 128))


def _threshold(gmax_pad):
    return pl.pallas_call(
        _thresh_block,
        grid=(1,),
        in_specs=[pl.BlockSpec((P, NBLK * 128), lambda j: (0, 0))],
        out_specs=pl.BlockSpec((P, 128), lambda j: (0, 0)),
        out_shape=jax.ShapeDtypeStruct((P, 128), jnp.float32),
    )(gmax_pad)


def _topk_kernel(simv_hbm, gmax_hbm, th_hbm, topk_hbm,
                 grow_v, tv, gids_v, gdma_v, sbuf, cand_v, cand_i, orow_v, sem):
    c = lax.axis_index("c")
    s = lax.axis_index("s")
    w = c * NS + s
    lanes = lax.iota(jnp.int32, 16)
    r0 = w * RPW

    pltpu.sync_copy(th_hbm.at[pl.ds(r0, RPW), :], tv)

    def _row(rl, _):
        r = r0 + rl
        pltpu.sync_copy(gmax_hbm.at[r], grow_v)
        trow = tv[rl, pl.ds(0, 16)]
        t = trow[0]

        for k in range(8):
            gids_v[pl.ds(k * 16, 16)] = k * 16 + lanes
        qcnt = jnp.int32(0)
        for k in range(NG // 16):
            g = grow_v[pl.ds(k * 128, 16)]
            q = g >= t
            qi = jnp.where(q, 1, 0).astype(jnp.int32)
            inc = plsc.cumsum(qi)
            pos = qcnt + inc - qi
            msk = q & (pos < CAP)
            plsc.store_scatter(gids_v, [pos], k * 16 + lanes, mask=msk)
            qcnt = qcnt + jnp.max(inc)
        qcnt = jnp.minimum(qcnt, jnp.int32(CAP))

        def _gd(k, _):
            gv = gids_v[pl.ds(k * 16, 16)]
            gdma_v[pl.ds(k * 16, 16)] = r * NG + gv
            return 0
        lax.fori_loop(0, 8, _gd, 0)
        pltpu.async_copy(simv_hbm.at[gdma_v], sbuf, sem).wait()

        def _zc(k, _):
            cand_v[pl.ds(k * 16, 16)] = jnp.full((16,), NEG, jnp.float32)
            cand_i[pl.ds(k * 16, 16)] = jnp.full((16,), 0x7FFFFFFF, jnp.int32)
            return 0
        lax.fori_loop(0, CAP // 16, _zc, 0)

        nslot8 = (qcnt + 7) // 8

        def _ext(j8, ccnt):
            gwin = gids_v[pl.ds(j8 * 8, 16)]
            for jr in range(8):
                slot = j8 * 8 + jr
                valid = slot < qcnt
                gsc = gwin[jr]
                for u in range(8):
                    v = sbuf[slot, pl.ds(u * 16, 16)]
                    mk = (v >= t) & valid
                    mi = jnp.where(mk, 1, 0).astype(jnp.int32)
                    inc = plsc.cumsum(mi)
                    pos = ccnt + inc - mi
                    mk2 = mk & (pos < CAP)
                    gl = gsc * GS + u * 16 + lanes
                    plsc.store_scatter(cand_v, [pos], v, mask=mk2)
                    plsc.store_scatter(cand_i, [pos], gl, mask=mk2)
                    ccnt = ccnt + jnp.max(inc)
            return ccnt
        ccnt = lax.fori_loop(0, nslot8, _ext, jnp.int32(0))
        ccnt = jnp.minimum(ccnt, jnp.int32(CAP))

        nj8 = (ccnt + 7) // 8

        def _rank(j8, rks):
            vwin = cand_v[pl.ds(j8 * 8, 16)]
            iwin = cand_i[pl.ds(j8 * 8, 16)]
            for jr in range(8):
                vj = vwin[jr]
                ij = iwin[jr]
                nr = []
                for u in range(8):
                    vi = cand_v[pl.ds(u * 16, 16)]
                    ii = cand_i[pl.ds(u * 16, 16)]
                    beat = (vj > vi) | ((vj == vi) & (ij < ii))
                    nr.append(rks[u] + jnp.where(beat, 1, 0).astype(jnp.int32))
                rks = tuple(nr)
            return rks
        rks = lax.fori_loop(0, nj8, _rank,
                            tuple(jnp.zeros((16,), jnp.int32) for _ in range(8)))

        def _zo(k, _):
            orow_v[pl.ds(k * 16, 16)] = jnp.zeros((16,), jnp.int32)
            return 0
        lax.fori_loop(0, 4, _zo, 0)
        for u in range(8):
            ii = cand_i[pl.ds(u * 16, 16)]
            rk = rks[u]
            plsc.store_scatter(orow_v, [rk], ii, mask=rk < TOP_K)
        pltpu.sync_copy(orow_v, topk_hbm.at[r])
        return 0
    lax.fori_loop(0, RPW, _row, 0)


def _topk_sc(simv, gmax_pad, thresh_b):
    mesh = plsc.VectorSubcoreMesh(core_axis_name="c", subcore_axis_name="s",
                                  num_cores=NC, num_subcores=NS)
    return pl.kernel(
        _topk_kernel,
        out_type=jax.ShapeDtypeStruct((P, 64), jnp.int32),
        mesh=mesh,
        compiler_params=pltpu.CompilerParams(needs_layout_passes=False),
        scratch_types=[
            pltpu.VMEM((NBLK * 128,), jnp.float32),
            pltpu.VMEM((RPW, 128), jnp.float32),
            pltpu.VMEM((CAP,), jnp.int32),
            pltpu.VMEM((CAP,), jnp.int32),
            pltpu.VMEM((CAP, GS), jnp.float32),
            pltpu.VMEM((CAP,), jnp.float32),
            pltpu.VMEM((CAP,), jnp.int32),
            pltpu.VMEM((64,), jnp.int32),
            pltpu.SemaphoreType.DMA,
        ],
    )(simv, gmax_pad, thresh_b)


def _unique_gather_kernel(flat_hbm, mem_hbm, inv_hbm, active_hbm,
                          ranks_sh, stage_sh, totals_sh,
                          flat_v, present_v, ranks_v, scat_idx, scat_val,
                          zeros_v, tot_v, inv_src, inv_out, gidx, gbuf, sem, sem2):
    c = lax.axis_index("c")
    s = lax.axis_index("s")
    w = c * NS + s
    lanes = lax.iota(jnp.int32, 16)
    ones16 = jnp.full((16,), 1, jnp.int32)
    zeros16 = jnp.zeros((16,), jnp.int32)
    m_base = s * M_PER_TILE

    def _zero(i, _):
        present_v[pl.ds(i * 16, 16)] = zeros16
        return 0
    lax.fori_loop(0, M_PER_TILE // 16, _zero, 0)

    def _zero2(i, _):
        zeros_v[pl.ds(i * 16, 16)] = zeros16
        return 0
    lax.fori_loop(0, ZCHUNK // 16, _zero2, 0)

    def _scan_chunk(ch, _):
        pltpu.sync_copy(flat_hbm.at[pl.ds(ch * FLAT_CHUNK, FLAT_CHUNK)], flat_v)
        def _mark(i, _):
            v = flat_v[pl.ds(i * 16, 16)]
            rel = v - m_base
            msk = (rel >= 0) & (rel < M_PER_TILE)
            plsc.store_scatter(present_v, [rel], ones16, mask=msk)
            return 0
        lax.fori_loop(0, FLAT_CHUNK // 16, _mark, 0)
        return 0
    lax.fori_loop(0, NS, _scan_chunk, 0)

    def _csum(i, carry):
        x = present_v[pl.ds(i * 16, 16)]
        inc = plsc.cumsum(x)
        ranks_v[pl.ds(i * 16, 16)] = inc - x + carry
        return carry + jnp.max(inc)
    total = lax.fori_loop(0, M_PER_TILE // 16, _csum, jnp.int32(0))
    flat_v[pl.ds(0, 16)] = jnp.full((16,), 1, jnp.int32) * total
    pltpu.sync_copy(flat_v.at[pl.ds(0, 16)], totals_sh.at[pl.ds(s * 16, 16)])
    plsc.subcore_barrier()

    pltpu.sync_copy(totals_sh, tot_v)
    off = jnp.int32(0)
    for r in range(NS):
        row = tot_v[pl.ds(r * 16, 16)]
        off = off + jnp.where(jnp.int32(r) < s, row[0], 0)

    def _shift(i, _):
        ranks_v[pl.ds(i * 16, 16)] = ranks_v[pl.ds(i * 16, 16)] + off
        return 0
    lax.fori_loop(0, M_PER_TILE // 16, _shift, 0)
    pltpu.sync_copy(ranks_v, ranks_sh.at[pl.ds(m_base, M_PER_TILE)])
    pltpu.sync_copy(zeros_v, stage_sh.at[pl.ds(s * ZCHUNK, ZCHUNK)])
    plsc.subcore_barrier()

    def _build(i, _):
        pr = present_v[pl.ds(i * 16, 16)]
        rk = ranks_v[pl.ds(i * 16, 16)]
        mv = m_base + i * 16 + lanes
        dump = N + lax.rem(i, jnp.int32(13)) * 16 + lanes
        idx = jnp.where(pr > 0, rk, dump)
        row = i // 8
        col = (i % 8) * 16
        scat_idx[row, pl.ds(col, 16)] = idx
        scat_val[row, pl.ds(col, 16)] = mv
        return 0
    lax.fori_loop(0, M_PER_TILE // 16, _build, 0)
    for j in range(M_PER_TILE // 128):
        pltpu.sync_copy(scat_val.at[j], stage_sh.at[scat_idx.at[j]])
    plsc.subcore_barrier()

    pltpu.sync_copy(flat_hbm.at[pl.ds(w * N_PER_W, N_PER_W)], inv_src)
    for j in range(N_PER_W // 80):
        pltpu.sync_copy(ranks_sh.at[inv_src.at[pl.ds(j * 80, 80)]],
                        inv_out.at[pl.ds(j * 80, 80)])
    pltpu.sync_copy(inv_out, inv_hbm.at[pl.ds(w * N_PER_W, N_PER_W)])

    sems = (sem, sem2)

    def _eissue(t, b):
        ch = (w + t * NW) * GROWS
        pltpu.sync_copy(stage_sh.at[pl.ds(ch, GROWS)], gidx.at[b])
        pltpu.async_copy(mem_hbm.at[gidx.at[b]], gbuf.at[b], sems[b])

    def _edrain(t, b):
        pltpu.make_async_copy(mem_hbm.at[gidx.at[b]], gbuf.at[b], sems[b]).wait()
        ch = (w + t * NW) * GROWS
        pltpu.sync_copy(gbuf.at[b], active_hbm.at[pl.ds(ch, GROWS), :])

    _eissue(0, 0)
    def _epair(i, _):
        _eissue(2 * i + 1, 1)
        _edrain(2 * i, 0)
        _eissue(2 * i + 2, 0)
        _edrain(2 * i + 1, 1)
        return 0
    lax.fori_loop(0, (NGC - 1) // 2, _epair, 0)
    _edrain(NGC - 1, 0)


def _unique_gather(flat, memory_nodes):
    mesh = plsc.VectorSubcoreMesh(core_axis_name="c", subcore_axis_name="s",
                                  num_cores=NC, num_subcores=NS)
    return pl.kernel(
        _unique_gather_kernel,
        out_type=(
            jax.ShapeDtypeStruct((N,), jnp.int32),
            jax.ShapeDtypeStruct((N, D), jnp.float32),
        ),
        mesh=mesh,
        compiler_params=pltpu.CompilerParams(needs_layout_passes=False),
        scratch_types=[
            pltpu.VMEM_SHARED((M,), jnp.int32),
            pltpu.VMEM_SHARED((STAGE,), jnp.int32),
            pltpu.VMEM_SHARED((NS * 16,), jnp.int32),
            pltpu.VMEM((FLAT_CHUNK,), jnp.int32),
            pltpu.VMEM((M_PER_TILE,), jnp.int32),
            pltpu.VMEM((M_PER_TILE,), jnp.int32),
            pltpu.VMEM((M_PER_TILE // 128, 128), jnp.int32),
            pltpu.VMEM((M_PER_TILE // 128, 128), jnp.int32),
            pltpu.VMEM((ZCHUNK,), jnp.int32),
            pltpu.VMEM((NS * 16,), jnp.int32),
            pltpu.VMEM((N_PER_W,), jnp.int32),
            pltpu.VMEM((N_PER_W,), jnp.int32),
            pltpu.VMEM((2, GROWS), jnp.int32),
            pltpu.VMEM((2, GROWS, D), jnp.float32),
            pltpu.SemaphoreType.DMA,
            pltpu.SemaphoreType.DMA,
        ],
    )(flat, memory_nodes)


def kernel(test_patches, memory_nodes_gpu):
    sim, gmax_pad = _similarity(test_patches, memory_nodes_gpu)
    thresh_b = _threshold(gmax_pad)
    simv = sim.reshape(P * NG, GS)
    topk_pad = _topk_sc(simv, gmax_pad, thresh_b)
    flat = topk_pad[:, :TOP_K].reshape(-1)
    inverse, active = _unique_gather(flat, memory_nodes_gpu)
    test_node_idx = jnp.repeat(jnp.arange(P, dtype=jnp.int32), TOP_K)
    edge_index = jnp.stack([inverse, test_node_idx], axis=0)
    return edge_index, active


# speedup vs baseline: 6.2769x; 1.3058x over previous
import jax
import jax.numpy as jnp
from jax import lax
from jax.experimental import pallas as pl
from jax.experimental.pallas import tpu as pltpu
from jax.experimental.pallas import tpu_sc as plsc

P = 1024
D = 512
M = 65536
TOP_K = 50
N = P * TOP_K
MB = 2048

NC = 2
NS = 16
NW = NC * NS

M_PER_TILE = M // NS
N_PER_W = N // NW
FLAT_CHUNK = N // NS
STAGE_PAD = 256
STAGE = N + STAGE_PAD
ZCHUNK = STAGE // NS
GROWS = 64
CHK = 32
M_PER_W = M // NW
NCHK = M_PER_W // CHK
ACT_PAD = N + NW * 64


GS = 128
NG = M // GS
GPB = MB // GS
NBLK = M // MB
NEG = -3.4e38
CAP = 128
RPW = P // NW


def _sim_block(lhs_ref, rhs_ref, out_ref, gmax_ref):
    s = jax.lax.dot_general(
        lhs_ref[...], rhs_ref[...],
        dimension_numbers=(((1,), (1,)), ((), ())),
        preferred_element_type=jnp.float32,
    )
    out_ref[...] = s
    gm = jnp.max(s.reshape(P, GPB, GS), axis=2)
    pad = jnp.full((P, 128 - GPB), NEG, jnp.float32)
    gmax_ref[...] = jnp.concatenate([gm, pad], axis=1)


def _similarity(test_patches, memory_nodes):
    return pl.pallas_call(
        _sim_block,
        grid=(NBLK,),
        in_specs=[
            pl.BlockSpec((P, D), lambda j: (0, 0)),
            pl.BlockSpec((MB, D), lambda j: (j, 0)),
        ],
        out_specs=[
            pl.BlockSpec((P, MB), lambda j: (0, j)),
            pl.BlockSpec((P, 128), lambda j: (0, j)),
        ],
        out_shape=[
            jax.ShapeDtypeStruct((P, M), jnp.float32),
            jax.ShapeDtypeStruct((P, NBLK * 128), jnp.float32),
        ],
    )(test_patches, memory_nodes)


def _thresh_block(gmax_ref, out_ref):
    w = gmax_ref[...]

    def _iter(i, w):
        m = jnp.max(w, axis=1, keepdims=True)
        return jnp.where(w == m, NEG, w)
    w = lax.fori_loop(0, TOP_K - 1, _iter, w)
    t = jnp.max(w, axis=1, keepdims=True)
    out_ref[...] = jnp.broadcast_to(t, (P, 128))


def _threshold(gmax_pad):
    return pl.pallas_call(
        _thresh_block,
        grid=(1,),
        in_specs=[pl.BlockSpec((P, NBLK * 128), lambda j: (0, 0))],
        out_specs=pl.BlockSpec((P, 128), lambda j: (0, 0)),
        out_shape=jax.ShapeDtypeStruct((P, 128), jnp.float32),
    )(gmax_pad)


def _topk_kernel(simv_hbm, gmax_hbm, th_hbm, topk_hbm,
                 grow_v, tv, gids_v, gdma_v, sbuf, cand_v, cand_i, orow_v, sem):
    c = lax.axis_index("c")
    s = lax.axis_index("s")
    w = c * NS + s
    lanes = lax.iota(jnp.int32, 16)
    r0 = w * RPW

    pltpu.sync_copy(th_hbm.at[pl.ds(r0, RPW), :], tv)

    def _row(rl, _):
        r = r0 + rl
        pltpu.sync_copy(gmax_hbm.at[r], grow_v)
        trow = tv[rl, pl.ds(0, 16)]
        t = trow[0]

        for k in range(8):
            gids_v[pl.ds(k * 16, 16)] = k * 16 + lanes
        qcnt = jnp.int32(0)
        for k in range(NG // 16):
            g = grow_v[pl.ds(k * 128, 16)]
            q = g >= t
            qi = jnp.where(q, 1, 0).astype(jnp.int32)
            inc = plsc.cumsum(qi)
            pos = qcnt + inc - qi
            msk = q & (pos < CAP)
            plsc.store_scatter(gids_v, [pos], k * 16 + lanes, mask=msk)
            qcnt = qcnt + jnp.max(inc)
        qcnt = jnp.minimum(qcnt, jnp.int32(CAP))

        def _gd(k, _):
            gv = gids_v[pl.ds(k * 16, 16)]
            gdma_v[pl.ds(k * 16, 16)] = r * NG + gv
            return 0
        lax.fori_loop(0, 8, _gd, 0)
        pltpu.async_copy(simv_hbm.at[gdma_v], sbuf, sem).wait()

        def _zc(k, _):
            cand_v[pl.ds(k * 16, 16)] = jnp.full((16,), NEG, jnp.float32)
            cand_i[pl.ds(k * 16, 16)] = jnp.full((16,), 0x7FFFFFFF, jnp.int32)
            return 0
        lax.fori_loop(0, CAP // 16, _zc, 0)

        nslot8 = (qcnt + 7) // 8

        def _ext(j8, ccnt):
            gwin = gids_v[pl.ds(j8 * 8, 16)]
            for jr in range(8):
                slot = j8 * 8 + jr
                valid = slot < qcnt
                gsc = gwin[jr]
                for u in range(8):
                    v = sbuf[slot, pl.ds(u * 16, 16)]
                    mk = (v >= t) & valid
                    mi = jnp.where(mk, 1, 0).astype(jnp.int32)
                    inc = plsc.cumsum(mi)
                    pos = ccnt + inc - mi
                    mk2 = mk & (pos < CAP)
                    gl = gsc * GS + u * 16 + lanes
                    plsc.store_scatter(cand_v, [pos], v, mask=mk2)
                    plsc.store_scatter(cand_i, [pos], gl, mask=mk2)
                    ccnt = ccnt + jnp.max(inc)
            return ccnt
        ccnt = lax.fori_loop(0, nslot8, _ext, jnp.int32(0))
        ccnt = jnp.minimum(ccnt, jnp.int32(CAP))

        nj8 = (ccnt + 7) // 8

        def _rank(j8, rks):
            vwin = cand_v[pl.ds(j8 * 8, 16)]
            iwin = cand_i[pl.ds(j8 * 8, 16)]
            for jr in range(8):
                vj = vwin[jr]
                ij = iwin[jr]
                nr = []
                for u in range(8):
                    vi = cand_v[pl.ds(u * 16, 16)]
                    ii = cand_i[pl.ds(u * 16, 16)]
                    beat = (vj > vi) | ((vj == vi) & (ij < ii))
                    nr.append(rks[u] + jnp.where(beat, 1, 0).astype(jnp.int32))
                rks = tuple(nr)
            return rks
        rks = lax.fori_loop(0, nj8, _rank,
                            tuple(jnp.zeros((16,), jnp.int32) for _ in range(8)))

        def _zo(k, _):
            orow_v[pl.ds(k * 16, 16)] = jnp.zeros((16,), jnp.int32)
            return 0
        lax.fori_loop(0, 4, _zo, 0)
        for u in range(8):
            ii = cand_i[pl.ds(u * 16, 16)]
            rk = rks[u]
            plsc.store_scatter(orow_v, [rk], ii, mask=rk < TOP_K)
        pltpu.sync_copy(orow_v, topk_hbm.at[r])
        return 0
    lax.fori_loop(0, RPW, _row, 0)


def _topk_sc(simv, gmax_pad, thresh_b):
    mesh = plsc.VectorSubcoreMesh(core_axis_name="c", subcore_axis_name="s",
                                  num_cores=NC, num_subcores=NS)
    return pl.kernel(
        _topk_kernel,
        out_type=jax.ShapeDtypeStruct((P, 64), jnp.int32),
        mesh=mesh,
        compiler_params=pltpu.CompilerParams(needs_layout_passes=False),
        scratch_types=[
            pltpu.VMEM((NBLK * 128,), jnp.float32),
            pltpu.VMEM((RPW, 128), jnp.float32),
            pltpu.VMEM((CAP,), jnp.int32),
            pltpu.VMEM((CAP,), jnp.int32),
            pltpu.VMEM((CAP, GS), jnp.float32),
            pltpu.VMEM((CAP,), jnp.float32),
            pltpu.VMEM((CAP,), jnp.int32),
            pltpu.VMEM((64,), jnp.int32),
            pltpu.SemaphoreType.DMA,
        ],
    )(simv, gmax_pad, thresh_b)


def _unique_gather_kernel(flat_hbm, mem_hbm, inv_hbm, active_hbm,
                          ranks_sh, totals_sh,
                          flat_v, present_v, ranks_v, tot_v, inv_src, inv_out,
                          sidx, mbuf, row0, zidx,
                          semra, semrb, semwa, semwb):
    c = lax.axis_index("c")
    s = lax.axis_index("s")
    w = c * NS + s
    lanes = lax.iota(jnp.int32, 16)
    ones16 = jnp.full((16,), 1, jnp.int32)
    zeros16 = jnp.zeros((16,), jnp.int32)
    m_base = s * M_PER_TILE

    def _zero(i, _):
        present_v[pl.ds(i * 16, 16)] = zeros16
        return 0
    lax.fori_loop(0, M_PER_TILE // 16, _zero, 0)

    def _scan_chunk(ch, _):
        pltpu.sync_copy(flat_hbm.at[pl.ds(ch * FLAT_CHUNK, FLAT_CHUNK)], flat_v)
        def _mark(i, _):
            v = flat_v[pl.ds(i * 16, 16)]
            rel = v - m_base
            msk = (rel >= 0) & (rel < M_PER_TILE)
            plsc.store_scatter(present_v, [rel], ones16, mask=msk)
            return 0
        lax.fori_loop(0, FLAT_CHUNK // 16, _mark, 0)
        return 0
    lax.fori_loop(0, NS, _scan_chunk, 0)

    def _csum(i, carry):
        x = present_v[pl.ds(i * 16, 16)]
        inc = plsc.cumsum(x)
        ranks_v[pl.ds(i * 16, 16)] = inc - x + carry
        return carry + jnp.max(inc)
    total = lax.fori_loop(0, M_PER_TILE // 16, _csum, jnp.int32(0))
    flat_v[pl.ds(0, 16)] = jnp.full((16,), 1, jnp.int32) * total
    pltpu.sync_copy(flat_v.at[pl.ds(0, 16)], totals_sh.at[pl.ds(s * 16, 16)])
    plsc.subcore_barrier()

    pltpu.sync_copy(totals_sh, tot_v)
    off = jnp.int32(0)
    nu = jnp.int32(0)
    for r in range(NS):
        row = tot_v[pl.ds(r * 16, 16)]
        off = off + jnp.where(jnp.int32(r) < s, row[0], 0)
        nu = nu + row[0]

    def _shift(i, _):
        ranks_v[pl.ds(i * 16, 16)] = ranks_v[pl.ds(i * 16, 16)] + off
        return 0
    lax.fori_loop(0, M_PER_TILE // 16, _shift, 0)
    pltpu.sync_copy(ranks_v, ranks_sh.at[pl.ds(m_base, M_PER_TILE)])
    plsc.subcore_barrier()

    pltpu.sync_copy(flat_hbm.at[pl.ds(w * N_PER_W, N_PER_W)], inv_src)
    for j in range(N_PER_W // 80):
        pltpu.sync_copy(ranks_sh.at[inv_src.at[pl.ds(j * 80, 80)]],
                        inv_out.at[pl.ds(j * 80, 80)])
    pltpu.sync_copy(inv_out, inv_hbm.at[pl.ds(w * N_PER_W, N_PER_W)])

    def _zz(k, _):
        zidx[pl.ds(k * 16, 16)] = jnp.zeros((16,), jnp.int32)
        return 0
    lax.fori_loop(0, GROWS // 16, _zz, 0)
    pltpu.async_copy(mem_hbm.at[zidx], row0, semra).wait()
    nfill = (N - nu + GROWS - 1) // GROWS
    nf_w = (nfill + NW - 1 - w) // NW

    def _fill(t, _):
        start = nu + (w + t * NW) * GROWS
        for k in range(GROWS // 16):
            zidx[pl.ds(k * 16, 16)] = start + k * 16 + lanes
        pltpu.async_copy(row0, active_hbm.at[zidx], semwa).wait()
        return 0
    lax.fori_loop(0, nf_w, _fill, 0)

    mw = s * M_PER_TILE + c * M_PER_W
    lw = c * M_PER_W
    rsems = (semra, semrb)
    wsems = (semwa, semwb)

    def _rd(t, b):
        pltpu.async_copy(mem_hbm.at[pl.ds(mw + t * CHK, CHK), :], mbuf.at[b], rsems[b])

    def _rdwait(t, b):
        pltpu.make_async_copy(mem_hbm.at[pl.ds(mw + t * CHK, CHK), :], mbuf.at[b], rsems[b]).wait()

    def _scat(t, b):
        for k in range(CHK // 16):
            pr = present_v[pl.ds(lw + t * CHK + k * 16, 16)]
            rk = ranks_v[pl.ds(lw + t * CHK + k * 16, 16)]
            dump = N + w * 64 + k * 16 + lanes
            sidx[b, pl.ds(k * 16, 16)] = jnp.where(pr > 0, rk, dump)
        pltpu.async_copy(mbuf.at[b], active_hbm.at[sidx.at[b]], wsems[b])

    def _scatwait(t, b):
        pltpu.make_async_copy(mbuf.at[b], active_hbm.at[sidx.at[b]], wsems[b]).wait()

    _rd(0, 0)
    _rd(1, 1)

    def _epair(i, _):
        a = 2 * i
        _rdwait(a, 0)
        _scat(a, 0)
        _rdwait(a + 1, 1)
        _scat(a + 1, 1)
        _scatwait(a, 0)
        _scatwait(a + 1, 1)

        @pl.when(i < NCHK // 2 - 1)
        def _():
            _rd(a + 2, 0)
            _rd(a + 3, 1)
        return 0
    lax.fori_loop(0, NCHK // 2, _epair, 0)


def _unique_gather(flat, memory_nodes):
    mesh = plsc.VectorSubcoreMesh(core_axis_name="c", subcore_axis_name="s",
                                  num_cores=NC, num_subcores=NS)
    return pl.kernel(
        _unique_gather_kernel,
        out_type=(
            jax.ShapeDtypeStruct((N,), jnp.int32),
            jax.ShapeDtypeStruct((ACT_PAD, D), jnp.float32),
        ),
        mesh=mesh,
        compiler_params=pltpu.CompilerParams(needs_layout_passes=False),
        scratch_types=[
            pltpu.VMEM_SHARED((M,), jnp.int32),
            pltpu.VMEM_SHARED((NS * 16,), jnp.int32),
            pltpu.VMEM((FLAT_CHUNK,), jnp.int32),
            pltpu.VMEM((M_PER_TILE,), jnp.int32),
            pltpu.VMEM((M_PER_TILE,), jnp.int32),
            pltpu.VMEM((NS * 16,), jnp.int32),
            pltpu.VMEM((N_PER_W,), jnp.int32),
            pltpu.VMEM((N_PER_W,), jnp.int32),
            pltpu.VMEM((2, CHK), jnp.int32),
            pltpu.VMEM((2, CHK, D), jnp.float32),
            pltpu.VMEM((GROWS, D), jnp.float32),
            pltpu.VMEM((GROWS,), jnp.int32),
            pltpu.SemaphoreType.DMA,
            pltpu.SemaphoreType.DMA,
            pltpu.SemaphoreType.DMA,
            pltpu.SemaphoreType.DMA,
        ],
    )(flat, memory_nodes)


def kernel(test_patches, memory_nodes_gpu):
    sim, gmax_pad = _similarity(test_patches, memory_nodes_gpu)
    thresh_b = _threshold(gmax_pad)
    simv = sim.reshape(P * NG, GS)
    topk_pad = _topk_sc(simv, gmax_pad, thresh_b)
    flat = topk_pad[:, :TOP_K].reshape(-1)
    inverse, active_pad = _unique_gather(flat, memory_nodes_gpu)
    active = active_pad[:N]
    test_node_idx = jnp.repeat(jnp.arange(P, dtype=jnp.int32), TOP_K)
    edge_index = jnp.stack([inverse, test_node_idx], axis=0)
    return edge_index, active


# speedup vs baseline: 6.3484x; 1.0114x over previous
import jax
import jax.numpy as jnp
from jax import lax
from jax.experimental import pallas as pl
from jax.experimental.pallas import tpu as pltpu
from jax.experimental.pallas import tpu_sc as plsc

P = 1024
D = 512
M = 65536
TOP_K = 50
N = P * TOP_K
MB = 2048

NC = 2
NS = 16
NW = NC * NS

M_PER_TILE = M // NS
N_PER_W = N // NW
FLAT_CHUNK = N // NS
STAGE_PAD = 256
STAGE = N + STAGE_PAD
ZCHUNK = STAGE // NS
GROWS = 64
CHK = 32
M_PER_W = M // NW
NCHK = M_PER_W // CHK
ACT_PAD = N + NW * 64


GS = 128
NG = M // GS
GPB = MB // GS
NBLK = M // MB
NEG = -3.4e38
CAP = 128
RPW = P // NW


def _sim_block(lhs_ref, rhs_ref, out_ref, gmax_ref):
    s = jax.lax.dot_general(
        lhs_ref[...], rhs_ref[...],
        dimension_numbers=(((1,), (1,)), ((), ())),
        preferred_element_type=jnp.float32,
    )
    out_ref[...] = s
    gm = jnp.max(s.reshape(P, GPB, GS), axis=2)
    pad = jnp.full((P, 128 - GPB), NEG, jnp.float32)
    gmax_ref[...] = jnp.concatenate([gm, pad], axis=1)


def _similarity(test_patches, memory_nodes):
    return pl.pallas_call(
        _sim_block,
        grid=(NBLK,),
        in_specs=[
            pl.BlockSpec((P, D), lambda j: (0, 0)),
            pl.BlockSpec((MB, D), lambda j: (j, 0)),
        ],
        out_specs=[
            pl.BlockSpec((P, MB), lambda j: (0, j)),
            pl.BlockSpec((P, 128), lambda j: (0, j)),
        ],
        out_shape=[
            jax.ShapeDtypeStruct((P, M), jnp.float32),
            jax.ShapeDtypeStruct((P, NBLK * 128), jnp.float32),
        ],
    )(test_patches, memory_nodes)


def _thresh_block(gmax_ref, out_ref):
    w = gmax_ref[...]

    def _iter(i, w):
        m = jnp.max(w, axis=1, keepdims=True)
        return jnp.where(w == m, NEG, w)
    w = lax.fori_loop(0, TOP_K - 1, _iter, w)
    t = jnp.max(w, axis=1, keepdims=True)
    out_ref[...] = jnp.broadcast_to(t, (P, 128))


def _threshold(gmax_pad):
    return pl.pallas_call(
        _thresh_block,
        grid=(1,),
        in_specs=[pl.BlockSpec((P, NBLK * 128), lambda j: (0, 0))],
        out_specs=pl.BlockSpec((P, 128), lambda j: (0, 0)),
        out_shape=jax.ShapeDtypeStruct((P, 128), jnp.float32),
    )(gmax_pad)


def _topk_kernel(simv_hbm, gmax_hbm, th_hbm, topk_hbm,
                 grow_v, tv, gids_v, gdma_v, sbuf, cand_v, cand_i, orow_v, sem):
    c = lax.axis_index("c")
    s = lax.axis_index("s")
    w = c * NS + s
    lanes = lax.iota(jnp.int32, 16)
    r0 = w * RPW

    pltpu.sync_copy(th_hbm.at[pl.ds(r0, RPW), :], tv)

    def _row(rl, _):
        r = r0 + rl

        @pl.when(rl % 8 == 0)
        def _():
            pltpu.sync_copy(gmax_hbm.at[pl.ds(pl.multiple_of(r, 8), 8), :], grow_v)
        rl8 = rl % 8
        trow = tv[rl, pl.ds(0, 16)]
        t = trow[0]

        for k in range(8):
            gids_v[pl.ds(k * 16, 16)] = k * 16 + lanes
        qcnt = jnp.int32(0)
        for k in range(NG // 16):
            g = grow_v[rl8, pl.ds(k * 128, 16)]
            q = g >= t
            qi = jnp.where(q, 1, 0).astype(jnp.int32)
            inc = plsc.cumsum(qi)
            pos = qcnt + inc - qi
            msk = q & (pos < CAP)
            plsc.store_scatter(gids_v, [pos], k * 16 + lanes, mask=msk)
            qcnt = qcnt + jnp.max(inc)
        qcnt = jnp.minimum(qcnt, jnp.int32(CAP))

        def _gd(k, _):
            gv = gids_v[pl.ds(k * 16, 16)]
            gdma_v[pl.ds(k * 16, 16)] = r * NG + gv
            return 0
        lax.fori_loop(0, 8, _gd, 0)
        gcopy = pltpu.async_copy(simv_hbm.at[gdma_v], sbuf, sem)

        def _zc(k, _):
            cand_v[pl.ds(k * 16, 16)] = jnp.full((16,), NEG, jnp.float32)
            cand_i[pl.ds(k * 16, 16)] = jnp.full((16,), 0x7FFFFFFF, jnp.int32)
            return 0
        lax.fori_loop(0, CAP // 16, _zc, 0)
        gcopy.wait()

        nslot8 = (qcnt + 7) // 8

        def _ext(j8, ccnt):
            gwin = gids_v[pl.ds(j8 * 8, 16)]
            for jr in range(8):
                slot = j8 * 8 + jr
                valid = slot < qcnt
                gsc = gwin[jr]
                for u in range(8):
                    v = sbuf[slot, pl.ds(u * 16, 16)]
                    mk = (v >= t) & valid
                    mi = jnp.where(mk, 1, 0).astype(jnp.int32)
                    inc = plsc.cumsum(mi)
                    pos = ccnt + inc - mi
                    mk2 = mk & (pos < CAP)
                    gl = gsc * GS + u * 16 + lanes
                    plsc.store_scatter(cand_v, [pos], v, mask=mk2)
                    plsc.store_scatter(cand_i, [pos], gl, mask=mk2)
                    ccnt = ccnt + jnp.max(inc)
            return ccnt
        ccnt = lax.fori_loop(0, nslot8, _ext, jnp.int32(0))
        ccnt = jnp.minimum(ccnt, jnp.int32(CAP))

        nj8 = (ccnt + 7) // 8

        def _rank(j8, rks):
            vwin = cand_v[pl.ds(j8 * 8, 16)]
            iwin = cand_i[pl.ds(j8 * 8, 16)]
            for jr in range(8):
                vj = vwin[jr]
                ij = iwin[jr]
                nr = []
                for u in range(8):
                    vi = cand_v[pl.ds(u * 16, 16)]
                    ii = cand_i[pl.ds(u * 16, 16)]
                    beat = (vj > vi) | ((vj == vi) & (ij < ii))
                    nr.append(rks[u] + jnp.where(beat, 1, 0).astype(jnp.int32))
                rks = tuple(nr)
            return rks
        rks = lax.fori_loop(0, nj8, _rank,
                            tuple(jnp.zeros((16,), jnp.int32) for _ in range(8)))

        def _zo(k, _):
            orow_v[pl.ds(k * 16, 16)] = jnp.zeros((16,), jnp.int32)
            return 0
        lax.fori_loop(0, 4, _zo, 0)
        for u in range(8):
            ii = cand_i[pl.ds(u * 16, 16)]
            rk = rks[u]
            plsc.store_scatter(orow_v, [rk], ii, mask=rk < TOP_K)
        pltpu.sync_copy(orow_v, topk_hbm.at[r])
        return 0
    lax.fori_loop(0, RPW, _row, 0)


def _topk_sc(simv, gmax_pad, thresh_b):
    mesh = plsc.VectorSubcoreMesh(core_axis_name="c", subcore_axis_name="s",
                                  num_cores=NC, num_subcores=NS)
    return pl.kernel(
        _topk_kernel,
        out_type=jax.ShapeDtypeStruct((P, 64), jnp.int32),
        mesh=mesh,
        compiler_params=pltpu.CompilerParams(needs_layout_passes=False),
        scratch_types=[
            pltpu.VMEM((8, NBLK * 128), jnp.float32),
            pltpu.VMEM((RPW, 128), jnp.float32),
            pltpu.VMEM((CAP,), jnp.int32),
            pltpu.VMEM((CAP,), jnp.int32),
            pltpu.VMEM((CAP, GS), jnp.float32),
            pltpu.VMEM((CAP,), jnp.float32),
            pltpu.VMEM((CAP,), jnp.int32),
            pltpu.VMEM((64,), jnp.int32),
            pltpu.SemaphoreType.DMA,
        ],
    )(simv, gmax_pad, thresh_b)


def _unique_gather_kernel(flat_hbm, mem_hbm, inv_hbm, active_hbm,
                          ranks_sh, totals_sh,
                          flat_v, present_v, ranks_v, tot_v, inv_src, inv_out,
                          sidx, mbuf, row0, zidx,
                          semra, semrb, semwa, semwb):
    c = lax.axis_index("c")
    s = lax.axis_index("s")
    w = c * NS + s
    lanes = lax.iota(jnp.int32, 16)
    ones16 = jnp.full((16,), 1, jnp.int32)
    zeros16 = jnp.zeros((16,), jnp.int32)
    m_base = s * M_PER_TILE

    def _zero(i, _):
        present_v[pl.ds(i * 16, 16)] = zeros16
        return 0
    lax.fori_loop(0, M_PER_TILE // 16, _zero, 0)

    def _scan_chunk(ch, _):
        pltpu.sync_copy(flat_hbm.at[pl.ds(ch * FLAT_CHUNK, FLAT_CHUNK)], flat_v)
        def _mark(i, _):
            v = flat_v[pl.ds(i * 16, 16)]
            rel = v - m_base
            msk = (rel >= 0) & (rel < M_PER_TILE)
            plsc.store_scatter(present_v, [rel], ones16, mask=msk)
            return 0
        lax.fori_loop(0, FLAT_CHUNK // 16, _mark, 0)
        return 0
    lax.fori_loop(0, NS, _scan_chunk, 0)

    def _csum(i, carry):
        x = present_v[pl.ds(i * 16, 16)]
        inc = plsc.cumsum(x)
        ranks_v[pl.ds(i * 16, 16)] = inc - x + carry
        return carry + jnp.max(inc)
    total = lax.fori_loop(0, M_PER_TILE // 16, _csum, jnp.int32(0))
    flat_v[pl.ds(0, 16)] = jnp.full((16,), 1, jnp.int32) * total
    pltpu.sync_copy(flat_v.at[pl.ds(0, 16)], totals_sh.at[pl.ds(s * 16, 16)])
    plsc.subcore_barrier()

    pltpu.sync_copy(totals_sh, tot_v)
    off = jnp.int32(0)
    nu = jnp.int32(0)
    for r in range(NS):
        row = tot_v[pl.ds(r * 16, 16)]
        off = off + jnp.where(jnp.int32(r) < s, row[0], 0)
        nu = nu + row[0]

    def _shift(i, _):
        ranks_v[pl.ds(i * 16, 16)] = ranks_v[pl.ds(i * 16, 16)] + off
        return 0
    lax.fori_loop(0, M_PER_TILE // 16, _shift, 0)
    pltpu.sync_copy(ranks_v, ranks_sh.at[pl.ds(m_base, M_PER_TILE)])
    plsc.subcore_barrier()

    pltpu.sync_copy(flat_hbm.at[pl.ds(w * N_PER_W, N_PER_W)], inv_src)
    for j in range(N_PER_W // 80):
        pltpu.sync_copy(ranks_sh.at[inv_src.at[pl.ds(j * 80, 80)]],
                        inv_out.at[pl.ds(j * 80, 80)])
    pltpu.sync_copy(inv_out, inv_hbm.at[pl.ds(w * N_PER_W, N_PER_W)])

    def _zz(k, _):
        zidx[pl.ds(k * 16, 16)] = jnp.zeros((16,), jnp.int32)
        return 0
    lax.fori_loop(0, GROWS // 16, _zz, 0)
    pltpu.async_copy(mem_hbm.at[zidx], row0, semra).wait()
    nfill = (N - nu + GROWS - 1) // GROWS
    nf_w = (nfill + NW - 1 - w) // NW

    def _fill(t, _):
        start = nu + (w + t * NW) * GROWS
        for k in range(GROWS // 16):
            zidx[pl.ds(k * 16, 16)] = start + k * 16 + lanes
        pltpu.async_copy(row0, active_hbm.at[zidx], semwa).wait()
        return 0
    lax.fori_loop(0, nf_w, _fill, 0)

    mw = s * M_PER_TILE + c * M_PER_W
    lw = c * M_PER_W
    rsems = (semra, semrb)
    wsems = (semwa, semwb)

    def _rd(t, b):
        pltpu.async_copy(mem_hbm.at[pl.ds(mw + t * CHK, CHK), :], mbuf.at[b], rsems[b])

    def _rdwait(t, b):
        pltpu.make_async_copy(mem_hbm.at[pl.ds(mw + t * CHK, CHK), :], mbuf.at[b], rsems[b]).wait()

    def _scat(t, b):
        for k in range(CHK // 16):
            pr = present_v[pl.ds(lw + t * CHK + k * 16, 16)]
            rk = ranks_v[pl.ds(lw + t * CHK + k * 16, 16)]
            dump = N + w * 64 + k * 16 + lanes
            sidx[b, pl.ds(k * 16, 16)] = jnp.where(pr > 0, rk, dump)
        pltpu.async_copy(mbuf.at[b], active_hbm.at[sidx.at[b]], wsems[b])

    def _scatwait(t, b):
        pltpu.make_async_copy(mbuf.at[b], active_hbm.at[sidx.at[b]], wsems[b]).wait()

    _rd(0, 0)
    _rd(1, 1)

    def _epair(i, _):
        a = 2 * i
        _rdwait(a, 0)
        _scat(a, 0)
        _rdwait(a + 1, 1)
        _scat(a + 1, 1)
        _scatwait(a, 0)
        _scatwait(a + 1, 1)

        @pl.when(i < NCHK // 2 - 1)
        def _():
            _rd(a + 2, 0)
            _rd(a + 3, 1)
        return 0
    lax.fori_loop(0, NCHK // 2, _epair, 0)


def _unique_gather(flat, memory_nodes):
    mesh = plsc.VectorSubcoreMesh(core_axis_name="c", subcore_axis_name="s",
                                  num_cores=NC, num_subcores=NS)
    return pl.kernel(
        _unique_gather_kernel,
        out_type=(
            jax.ShapeDtypeStruct((N,), jnp.int32),
            jax.ShapeDtypeStruct((ACT_PAD, D), jnp.float32),
        ),
        mesh=mesh,
        compiler_params=pltpu.CompilerParams(needs_layout_passes=False),
        scratch_types=[
            pltpu.VMEM_SHARED((M,), jnp.int32),
            pltpu.VMEM_SHARED((NS * 16,), jnp.int32),
            pltpu.VMEM((FLAT_CHUNK,), jnp.int32),
            pltpu.VMEM((M_PER_TILE,), jnp.int32),
            pltpu.VMEM((M_PER_TILE,), jnp.int32),
            pltpu.VMEM((NS * 16,), jnp.int32),
            pltpu.VMEM((N_PER_W,), jnp.int32),
            pltpu.VMEM((N_PER_W,), jnp.int32),
            pltpu.VMEM((2, CHK), jnp.int32),
            pltpu.VMEM((2, CHK, D), jnp.float32),
            pltpu.VMEM((GROWS, D), jnp.float32),
            pltpu.VMEM((GROWS,), jnp.int32),
            pltpu.SemaphoreType.DMA,
            pltpu.SemaphoreType.DMA,
            pltpu.SemaphoreType.DMA,
            pltpu.SemaphoreType.DMA,
        ],
    )(flat, memory_nodes)


def kernel(test_patches, memory_nodes_gpu):
    sim, gmax_pad = _similarity(test_patches, memory_nodes_gpu)
    thresh_b = _threshold(gmax_pad)
    simv = sim.reshape(P * NG, GS)
    topk_pad = _topk_sc(simv, gmax_pad, thresh_b)
    flat = topk_pad[:, :TOP_K].reshape(-1)
    inverse, active_pad = _unique_gather(flat, memory_nodes_gpu)
    active = active_pad[:N]
    test_node_idx = jnp.repeat(jnp.arange(P, dtype=jnp.int32), TOP_K)
    edge_index = jnp.stack([inverse, test_node_idx], axis=0)
    return edge_index, active
